# baseline (speedup 1.0000x reference)
import functools

import jax
import jax.numpy as jnp
from jax import lax
from jax.experimental import pallas as pl
from jax.experimental.pallas import tpu as pltpu
from jax.experimental.pallas import tpu_sc as plsc

N = 10000
E = 160000
DN = 128
DG = 64

NC = 2
NS = 16
NW = NC * NS

CHUNK = 125
EDGES_PER_W = E // NW
NCH = EDGES_PER_W // CHUNK

_gather_mesh = plsc.VectorSubcoreMesh(core_axis_name="c", subcore_axis_name="s")
_sc_params = pltpu.CompilerParams(use_tc_tiling_on_sc=False)


@functools.partial(
    pl.kernel,
    out_type=[jax.ShapeDtypeStruct((E, DN), jnp.bfloat16),
              jax.ShapeDtypeStruct((E, DN), jnp.bfloat16)],
    mesh=_gather_mesh,
    compiler_params=_sc_params,
    scratch_types=[
        pltpu.VMEM((NCH, CHUNK), jnp.int32),
        pltpu.VMEM((NCH, CHUNK), jnp.int32),
        pltpu.VMEM((CHUNK, DN), jnp.bfloat16),
        pltpu.VMEM((CHUNK, DN), jnp.bfloat16),
        pltpu.SemaphoreType.DMA,
        pltpu.SemaphoreType.DMA,
    ],
)
def _gather_rows(nodes_hbm, sidx_hbm, didx_hbm, sout_hbm, dout_hbm,
                 sidx_v, didx_v, srows_v, drows_v, sem1, sem2):
    wid = lax.axis_index("s") * NC + lax.axis_index("c")
    pltpu.sync_copy(sidx_hbm.at[wid], sidx_v)
    pltpu.sync_copy(didx_hbm.at[wid], didx_v)
    base = wid * EDGES_PER_W

    @pl.loop(0, NCH)
    def _chunk(c):
        cp1 = pltpu.async_copy(nodes_hbm.at[sidx_v.at[c]], srows_v, sem1)
        cp2 = pltpu.async_copy(nodes_hbm.at[didx_v.at[c]], drows_v, sem2)
        cp1.wait()
        pltpu.sync_copy(srows_v, sout_hbm.at[pl.ds(base + c * CHUNK, CHUNK)])
        cp2.wait()
        pltpu.sync_copy(drows_v, dout_hbm.at[pl.ds(base + c * CHUNK, CHUNK)])


S_ROWS_PER_TILE = N // NS


@functools.partial(
    pl.kernel,
    out_type=jax.ShapeDtypeStruct((NC, N, DN), jnp.float32),
    mesh=_gather_mesh,
    compiler_params=_sc_params,
    scratch_types=[
        pltpu.VMEM((NCH, CHUNK), jnp.int32),
        pltpu.VMEM((NCH, CHUNK), jnp.int32),
        pltpu.VMEM((CHUNK, DN), jnp.float32),
        pltpu.VMEM_SHARED((N, DN), jnp.float32),
    ],
)
def _scatter_add(feat_hbm, sidx_hbm, didx_hbm, zeros_hbm, out_hbm,
                 sidx_v, didx_v, feat_v, agg_sh):
    cid = lax.axis_index("c")
    sid = lax.axis_index("s")
    wid = sid * NC + cid
    pltpu.sync_copy(sidx_hbm.at[wid], sidx_v)
    pltpu.sync_copy(didx_hbm.at[wid], didx_v)
    rslice = pl.ds(sid * S_ROWS_PER_TILE, S_ROWS_PER_TILE)
    pltpu.sync_copy(zeros_hbm.at[rslice], agg_sh.at[rslice])
    plsc.subcore_barrier()
    base = wid * EDGES_PER_W

    @pl.loop(0, NCH)
    def _chunk(c):
        pltpu.sync_copy(feat_hbm.at[pl.ds(base + c * CHUNK, CHUNK)], feat_v)
        pltpu.sync_copy(feat_v, agg_sh.at[sidx_v.at[c]], add=True)
        pltpu.sync_copy(feat_v, agg_sh.at[didx_v.at[c]], add=True)

    plsc.subcore_barrier()
    pltpu.sync_copy(agg_sh.at[rslice], out_hbm.at[cid, rslice])


EB = 2000


def _edge_mlp_body(e_ref, s_ref, d_ref, gl_ref,
                   w0e, w0s, w0d, w0g, b0, w1, b1, w2, b2, gam, bet,
                   out_ref, sum_ref):
    i = pl.program_id(0)
    x = jnp.dot(e_ref[...], w0e[...], preferred_element_type=jnp.float32)
    x += jnp.dot(s_ref[...], w0s[...], preferred_element_type=jnp.float32)
    x += jnp.dot(d_ref[...], w0d[...], preferred_element_type=jnp.float32)
    gbb = jnp.dot(gl_ref[...], w0g[...],
                  preferred_element_type=jnp.float32) + b0[...]
    h = jax.nn.relu(x + gbb)
    h = jax.nn.relu(jnp.dot(h.astype(jnp.bfloat16), w1[...],
                            preferred_element_type=jnp.float32) + b1[...])
    h = jax.nn.sigmoid(jnp.dot(h.astype(jnp.bfloat16), w2[...],
                               preferred_element_type=jnp.float32) + b2[...])
    mu = jnp.mean(h, axis=-1, keepdims=True)
    hc = h - mu
    var = jnp.mean(hc * hc, axis=-1, keepdims=True)
    y = hc * lax.rsqrt(var + 1e-3) * gam[...] + bet[...]
    out_ref[...] = y

    @pl.when(i == 0)
    def _init():
        sum_ref[...] = jnp.zeros_like(sum_ref)

    sum_ref[...] += jnp.sum(y, axis=0, keepdims=True)


def _edge_mlp(edges, src_feat, dst_feat, globals_, p):
    w0 = p["W0"]

    def rep(w):
        return pl.BlockSpec(w.shape, lambda i: (0,) * w.ndim)

    grid = E // EB
    return pl.pallas_call(
        _edge_mlp_body,
        grid=(grid,),
        in_specs=[
            pl.BlockSpec((EB, 16), lambda i: (i, 0)),
            pl.BlockSpec((EB, 128), lambda i: (i, 0)),
            pl.BlockSpec((EB, 128), lambda i: (i, 0)),
            pl.BlockSpec((1, DG), lambda i: (0, 0)),
            rep(w0[:16]), rep(w0[16:144]), rep(w0[144:272]), rep(w0[272:336]),
            pl.BlockSpec((1, 256), lambda i: (0, 0)),
            pl.BlockSpec((256, 256), lambda i: (0, 0)),
            pl.BlockSpec((1, 256), lambda i: (0, 0)),
            pl.BlockSpec((256, 128), lambda i: (0, 0)),
            pl.BlockSpec((1, 128), lambda i: (0, 0)),
            pl.BlockSpec((1, 128), lambda i: (0, 0)),
            pl.BlockSpec((1, 128), lambda i: (0, 0)),
        ],
        out_specs=[
            pl.BlockSpec((EB, 128), lambda i: (i, 0)),
            pl.BlockSpec((1, 128), lambda i: (0, 0)),
        ],
        out_shape=[
            jax.ShapeDtypeStruct((E, 128), jnp.float32),
            jax.ShapeDtypeStruct((1, 128), jnp.float32),
        ],
    )(edges.astype(jnp.bfloat16), src_feat, dst_feat,
      globals_.astype(jnp.bfloat16),
      w0[:16].astype(jnp.bfloat16), w0[16:144].astype(jnp.bfloat16),
      w0[144:272].astype(jnp.bfloat16), w0[272:336].astype(jnp.bfloat16),
      p["b0"].reshape(1, -1),
      p["W1"].astype(jnp.bfloat16), p["b1"].reshape(1, -1),
      p["W2"].astype(jnp.bfloat16), p["b2"].reshape(1, -1),
      p["g"].reshape(1, -1), p["beta"].reshape(1, -1))


NB = 2000


def _node_mlp_body(n_ref, a_ref, gl_ref, esum_ref,
                   w0n, w0g, w0a, b0, w1, b1, w2, b2, gam, bet,
                   gw0g, gw0n, gw0e, gb0, gw1, gb1, gw2, gb2, ggam, gbet,
                   out_ref, gout_ref, nsum):
    i = pl.program_id(0)
    agg = (a_ref[0] + a_ref[1]).astype(jnp.bfloat16)
    x = jnp.dot(n_ref[...], w0n[...], preferred_element_type=jnp.float32)
    x += jnp.dot(agg, w0a[...], preferred_element_type=jnp.float32)
    gbb = jnp.dot(gl_ref[...], w0g[...],
                  preferred_element_type=jnp.float32) + b0[...]
    h = jax.nn.relu(x + gbb)
    h = jax.nn.relu(jnp.dot(h.astype(jnp.bfloat16), w1[...],
                            preferred_element_type=jnp.float32) + b1[...])
    h = jax.nn.sigmoid(jnp.dot(h.astype(jnp.bfloat16), w2[...],
                               preferred_element_type=jnp.float32) + b2[...])
    mu = jnp.mean(h, axis=-1, keepdims=True)
    hc = h - mu
    var = jnp.mean(hc * hc, axis=-1, keepdims=True)
    y = hc * lax.rsqrt(var + 1e-3) * gam[...] + bet[...]
    out_ref[...] = y

    @pl.when(i == 0)
    def _init():
        nsum[...] = jnp.zeros_like(nsum)

    nsum[...] += jnp.sum(y, axis=0, keepdims=True)

    @pl.when(i == pl.num_programs(0) - 1)
    def _globals():
        nm = nsum[...] / N
        em = esum_ref[...] / E
        gx = jnp.dot(gl_ref[...], gw0g[...], preferred_element_type=jnp.float32)
        gx += jnp.dot(nm, gw0n[...], preferred_element_type=jnp.float32)
        gx += jnp.dot(em, gw0e[...], preferred_element_type=jnp.float32)
        gh = jax.nn.relu(gx + gb0[...])
        gh = jax.nn.relu(jnp.dot(gh, gw1[...], preferred_element_type=jnp.float32) + gb1[...])
        gh = jax.nn.sigmoid(jnp.dot(gh, gw2[...], preferred_element_type=jnp.float32) + gb2[...])
        gmu = jnp.mean(gh, axis=-1, keepdims=True)
        ghc = gh - gmu
        gvar = jnp.mean(ghc * ghc, axis=-1, keepdims=True)
        gout_ref[...] = ghc * lax.rsqrt(gvar + 1e-3) * ggam[...] + gbet[...]


def _node_mlp(nodes, agg2, globals_, esum, p, gp):
    w0 = p["W0"]
    gw0 = gp["W0"]

    def rep(w):
        return pl.BlockSpec(w.shape, lambda i: (0,) * w.ndim)

    grid = N // NB
    return pl.pallas_call(
        _node_mlp_body,
        grid=(grid,),
        in_specs=[
            pl.BlockSpec((NB, DN), lambda i: (i, 0)),
            pl.BlockSpec((NC, NB, DN), lambda i: (0, i, 0)),
            pl.BlockSpec((1, DG), lambda i: (0, 0)),
            pl.BlockSpec((1, 128), lambda i: (0, 0)),
            rep(w0[:128]), rep(w0[128:192]), rep(w0[192:320]),
            rep(p["b0"].reshape(1, -1)), rep(p["W1"]), rep(p["b1"].reshape(1, -1)),
            rep(p["W2"]), rep(p["b2"].reshape(1, -1)),
            rep(p["g"].reshape(1, -1)), rep(p["beta"].reshape(1, -1)),
            rep(gw0[:64]), rep(gw0[64:192]), rep(gw0[192:320]),
            rep(gp["b0"].reshape(1, -1)), rep(gp["W1"]), rep(gp["b1"].reshape(1, -1)),
            rep(gp["W2"]), rep(gp["b2"].reshape(1, -1)),
            rep(gp["g"].reshape(1, -1)), rep(gp["beta"].reshape(1, -1)),
        ],
        out_specs=[
            pl.BlockSpec((NB, 128), lambda i: (i, 0)),
            pl.BlockSpec((1, DG), lambda i: (0, 0)),
        ],
        out_shape=[
            jax.ShapeDtypeStruct((N, 128), jnp.float32),
            jax.ShapeDtypeStruct((1, DG), jnp.float32),
        ],
        scratch_shapes=[pltpu.VMEM((1, 128), jnp.float32)],
    )(nodes.astype(jnp.bfloat16), agg2, globals_, esum,
      w0[:128].astype(jnp.bfloat16), w0[128:192], w0[192:320].astype(jnp.bfloat16),
      p["b0"].reshape(1, -1), p["W1"].astype(jnp.bfloat16), p["b1"].reshape(1, -1),
      p["W2"].astype(jnp.bfloat16), p["b2"].reshape(1, -1),
      p["g"].reshape(1, -1), p["beta"].reshape(1, -1),
      gw0[:64], gw0[64:192], gw0[192:320],
      gp["b0"].reshape(1, -1), gp["W1"], gp["b1"].reshape(1, -1),
      gp["W2"], gp["b2"].reshape(1, -1),
      gp["g"].reshape(1, -1), gp["beta"].reshape(1, -1))


def kernel(nodes, edges, globals_, edge_index, params):
    ei = edge_index.astype(jnp.int32)
    sidx = ei[:, 0].reshape(NW, NCH, CHUNK)
    didx = ei[:, 1].reshape(NW, NCH, CHUNK)
    src_feat, dst_feat = _gather_rows(nodes.astype(jnp.bfloat16), sidx, didx)
    edges_new, esum = _edge_mlp(edges, src_feat, dst_feat, globals_,
                                params["edge"])
    zeros = jnp.zeros((N, DN), jnp.float32)
    agg2 = _scatter_add(edges_new, sidx, didx, zeros)
    nodes_new, globals_new = _node_mlp(nodes, agg2, globals_, esum,
                                       params["node"], params["global"])
    return nodes_new, edges_new, globals_new


# speedup vs baseline: 1.5756x; 1.5756x over previous
import functools

import jax
import jax.numpy as jnp
from jax import lax
from jax.experimental import pallas as pl
from jax.experimental.pallas import tpu as pltpu
from jax.experimental.pallas import tpu_sc as plsc

N = 10000
E = 160000
DN = 128
DG = 64

NC = 2
NS = 16
NW = NC * NS

CHUNK = 125
EDGES_PER_W = E // NW
NCH = EDGES_PER_W // CHUNK

_gather_mesh = plsc.VectorSubcoreMesh(core_axis_name="c", subcore_axis_name="s")
_sc_params = pltpu.CompilerParams(use_tc_tiling_on_sc=False)


@functools.partial(
    pl.kernel,
    out_type=[jax.ShapeDtypeStruct((E, DN), jnp.float32),
              jax.ShapeDtypeStruct((E, DN), jnp.float32)],
    mesh=_gather_mesh,
    compiler_params=_sc_params,
    scratch_types=[
        pltpu.VMEM((NCH, CHUNK), jnp.int32),
        pltpu.VMEM((NCH, CHUNK), jnp.int32),
        pltpu.VMEM((CHUNK, DN), jnp.float32),
        pltpu.VMEM((CHUNK, DN), jnp.float32),
        pltpu.SemaphoreType.DMA,
        pltpu.SemaphoreType.DMA,
    ],
)
def _gather_rows(nodes_hbm, sidx_hbm, didx_hbm, sout_hbm, dout_hbm,
                 sidx_v, didx_v, srows_v, drows_v, sem1, sem2):
    wid = lax.axis_index("s") * NC + lax.axis_index("c")
    pltpu.sync_copy(sidx_hbm.at[wid], sidx_v)
    pltpu.sync_copy(didx_hbm.at[wid], didx_v)
    base = wid * EDGES_PER_W

    @pl.loop(0, NCH)
    def _chunk(c):
        cp1 = pltpu.async_copy(nodes_hbm.at[sidx_v.at[c]], srows_v, sem1)
        cp2 = pltpu.async_copy(nodes_hbm.at[didx_v.at[c]], drows_v, sem2)
        cp1.wait()
        pltpu.sync_copy(srows_v, sout_hbm.at[pl.ds(base + c * CHUNK, CHUNK)])
        cp2.wait()
        pltpu.sync_copy(drows_v, dout_hbm.at[pl.ds(base + c * CHUNK, CHUNK)])


S_ROWS_PER_TILE = N // NS


@functools.partial(
    pl.kernel,
    out_type=jax.ShapeDtypeStruct((NC, N, DN), jnp.float32),
    mesh=_gather_mesh,
    compiler_params=_sc_params,
    scratch_types=[
        pltpu.VMEM((NCH, CHUNK), jnp.int32),
        pltpu.VMEM((NCH, CHUNK), jnp.int32),
        pltpu.VMEM((CHUNK, DN), jnp.float32),
        pltpu.VMEM_SHARED((N, DN), jnp.float32),
    ],
)
def _scatter_add(feat_hbm, sidx_hbm, didx_hbm, zeros_hbm, out_hbm,
                 sidx_v, didx_v, feat_v, agg_sh):
    cid = lax.axis_index("c")
    sid = lax.axis_index("s")
    wid = sid * NC + cid
    pltpu.sync_copy(sidx_hbm.at[wid], sidx_v)
    pltpu.sync_copy(didx_hbm.at[wid], didx_v)
    rslice = pl.ds(sid * S_ROWS_PER_TILE, S_ROWS_PER_TILE)
    pltpu.sync_copy(zeros_hbm.at[rslice], agg_sh.at[rslice])
    plsc.subcore_barrier()
    base = wid * EDGES_PER_W

    @pl.loop(0, NCH)
    def _chunk(c):
        pltpu.sync_copy(feat_hbm.at[pl.ds(base + c * CHUNK, CHUNK)], feat_v)
        pltpu.sync_copy(feat_v, agg_sh.at[sidx_v.at[c]], add=True)
        pltpu.sync_copy(feat_v, agg_sh.at[didx_v.at[c]], add=True)

    plsc.subcore_barrier()
    pltpu.sync_copy(agg_sh.at[rslice], out_hbm.at[cid, rslice])


EB = 2000


def _edge_mlp_body(e_ref, s_ref, d_ref, gl_ref,
                   w0e, w0s, w0d, w0g, b0, w1, b1, w2, b2, gam, bet,
                   out_ref, sum_ref):
    i = pl.program_id(0)
    x = jnp.dot(e_ref[...], w0e[...], preferred_element_type=jnp.float32)
    x += jnp.dot(s_ref[...].astype(jnp.bfloat16), w0s[...],
                 preferred_element_type=jnp.float32)
    x += jnp.dot(d_ref[...].astype(jnp.bfloat16), w0d[...],
                 preferred_element_type=jnp.float32)
    gbb = jnp.dot(gl_ref[...], w0g[...],
                  preferred_element_type=jnp.float32) + b0[...]
    h = jax.nn.relu(x + gbb)
    h = jax.nn.relu(jnp.dot(h.astype(jnp.bfloat16), w1[...],
                            preferred_element_type=jnp.float32) + b1[...])
    h = jax.nn.sigmoid(jnp.dot(h.astype(jnp.bfloat16), w2[...],
                               preferred_element_type=jnp.float32) + b2[...])
    mu = jnp.mean(h, axis=-1, keepdims=True)
    hc = h - mu
    var = jnp.mean(hc * hc, axis=-1, keepdims=True)
    y = hc * lax.rsqrt(var + 1e-3) * gam[...] + bet[...]
    out_ref[...] = y

    @pl.when(i == 0)
    def _init():
        sum_ref[...] = jnp.zeros_like(sum_ref)

    sum_ref[...] += jnp.sum(y, axis=0, keepdims=True)


def _edge_mlp(edges, src_feat, dst_feat, globals_, p):
    w0 = p["W0"]

    def rep(w):
        return pl.BlockSpec(w.shape, lambda i: (0,) * w.ndim)

    grid = E // EB
    return pl.pallas_call(
        _edge_mlp_body,
        grid=(grid,),
        in_specs=[
            pl.BlockSpec((EB, 16), lambda i: (i, 0)),
            pl.BlockSpec((EB, 128), lambda i: (i, 0)),
            pl.BlockSpec((EB, 128), lambda i: (i, 0)),
            pl.BlockSpec((1, DG), lambda i: (0, 0)),
            rep(w0[:16]), rep(w0[16:144]), rep(w0[144:272]), rep(w0[272:336]),
            pl.BlockSpec((1, 256), lambda i: (0, 0)),
            pl.BlockSpec((256, 256), lambda i: (0, 0)),
            pl.BlockSpec((1, 256), lambda i: (0, 0)),
            pl.BlockSpec((256, 128), lambda i: (0, 0)),
            pl.BlockSpec((1, 128), lambda i: (0, 0)),
            pl.BlockSpec((1, 128), lambda i: (0, 0)),
            pl.BlockSpec((1, 128), lambda i: (0, 0)),
        ],
        out_specs=[
            pl.BlockSpec((EB, 128), lambda i: (i, 0)),
            pl.BlockSpec((1, 128), lambda i: (0, 0)),
        ],
        out_shape=[
            jax.ShapeDtypeStruct((E, 128), jnp.float32),
            jax.ShapeDtypeStruct((1, 128), jnp.float32),
        ],
    )(edges.astype(jnp.bfloat16), src_feat, dst_feat,
      globals_.astype(jnp.bfloat16),
      w0[:16].astype(jnp.bfloat16), w0[16:144].astype(jnp.bfloat16),
      w0[144:272].astype(jnp.bfloat16), w0[272:336].astype(jnp.bfloat16),
      p["b0"].reshape(1, -1),
      p["W1"].astype(jnp.bfloat16), p["b1"].reshape(1, -1),
      p["W2"].astype(jnp.bfloat16), p["b2"].reshape(1, -1),
      p["g"].reshape(1, -1), p["beta"].reshape(1, -1))


NB = 2000


def _node_mlp_body(n_ref, a_ref, gl_ref, esum_ref,
                   w0n, w0g, w0a, b0, w1, b1, w2, b2, gam, bet,
                   gw0g, gw0n, gw0e, gb0, gw1, gb1, gw2, gb2, ggam, gbet,
                   out_ref, gout_ref, nsum):
    i = pl.program_id(0)
    agg = (a_ref[0] + a_ref[1]).astype(jnp.bfloat16)
    x = jnp.dot(n_ref[...], w0n[...], preferred_element_type=jnp.float32)
    x += jnp.dot(agg, w0a[...], preferred_element_type=jnp.float32)
    gbb = jnp.dot(gl_ref[...], w0g[...],
                  preferred_element_type=jnp.float32) + b0[...]
    h = jax.nn.relu(x + gbb)
    h = jax.nn.relu(jnp.dot(h.astype(jnp.bfloat16), w1[...],
                            preferred_element_type=jnp.float32) + b1[...])
    h = jax.nn.sigmoid(jnp.dot(h.astype(jnp.bfloat16), w2[...],
                               preferred_element_type=jnp.float32) + b2[...])
    mu = jnp.mean(h, axis=-1, keepdims=True)
    hc = h - mu
    var = jnp.mean(hc * hc, axis=-1, keepdims=True)
    y = hc * lax.rsqrt(var + 1e-3) * gam[...] + bet[...]
    out_ref[...] = y

    @pl.when(i == 0)
    def _init():
        nsum[...] = jnp.zeros_like(nsum)

    nsum[...] += jnp.sum(y, axis=0, keepdims=True)

    @pl.when(i == pl.num_programs(0) - 1)
    def _globals():
        nm = nsum[...] / N
        em = esum_ref[...] / E
        gx = jnp.dot(gl_ref[...], gw0g[...], preferred_element_type=jnp.float32)
        gx += jnp.dot(nm, gw0n[...], preferred_element_type=jnp.float32)
        gx += jnp.dot(em, gw0e[...], preferred_element_type=jnp.float32)
        gh = jax.nn.relu(gx + gb0[...])
        gh = jax.nn.relu(jnp.dot(gh, gw1[...], preferred_element_type=jnp.float32) + gb1[...])
        gh = jax.nn.sigmoid(jnp.dot(gh, gw2[...], preferred_element_type=jnp.float32) + gb2[...])
        gmu = jnp.mean(gh, axis=-1, keepdims=True)
        ghc = gh - gmu
        gvar = jnp.mean(ghc * ghc, axis=-1, keepdims=True)
        gout_ref[...] = ghc * lax.rsqrt(gvar + 1e-3) * ggam[...] + gbet[...]


def _node_mlp(nodes, agg2, globals_, esum, p, gp):
    w0 = p["W0"]
    gw0 = gp["W0"]

    def rep(w):
        return pl.BlockSpec(w.shape, lambda i: (0,) * w.ndim)

    grid = N // NB
    return pl.pallas_call(
        _node_mlp_body,
        grid=(grid,),
        in_specs=[
            pl.BlockSpec((NB, DN), lambda i: (i, 0)),
            pl.BlockSpec((NC, NB, DN), lambda i: (0, i, 0)),
            pl.BlockSpec((1, DG), lambda i: (0, 0)),
            pl.BlockSpec((1, 128), lambda i: (0, 0)),
            rep(w0[:128]), rep(w0[128:192]), rep(w0[192:320]),
            rep(p["b0"].reshape(1, -1)), rep(p["W1"]), rep(p["b1"].reshape(1, -1)),
            rep(p["W2"]), rep(p["b2"].reshape(1, -1)),
            rep(p["g"].reshape(1, -1)), rep(p["beta"].reshape(1, -1)),
            rep(gw0[:64]), rep(gw0[64:192]), rep(gw0[192:320]),
            rep(gp["b0"].reshape(1, -1)), rep(gp["W1"]), rep(gp["b1"].reshape(1, -1)),
            rep(gp["W2"]), rep(gp["b2"].reshape(1, -1)),
            rep(gp["g"].reshape(1, -1)), rep(gp["beta"].reshape(1, -1)),
        ],
        out_specs=[
            pl.BlockSpec((NB, 128), lambda i: (i, 0)),
            pl.BlockSpec((1, DG), lambda i: (0, 0)),
        ],
        out_shape=[
            jax.ShapeDtypeStruct((N, 128), jnp.float32),
            jax.ShapeDtypeStruct((1, DG), jnp.float32),
        ],
        scratch_shapes=[pltpu.VMEM((1, 128), jnp.float32)],
    )(nodes.astype(jnp.bfloat16), agg2, globals_, esum,
      w0[:128].astype(jnp.bfloat16), w0[128:192], w0[192:320].astype(jnp.bfloat16),
      p["b0"].reshape(1, -1), p["W1"].astype(jnp.bfloat16), p["b1"].reshape(1, -1),
      p["W2"].astype(jnp.bfloat16), p["b2"].reshape(1, -1),
      p["g"].reshape(1, -1), p["beta"].reshape(1, -1),
      gw0[:64], gw0[64:192], gw0[192:320],
      gp["b0"].reshape(1, -1), gp["W1"], gp["b1"].reshape(1, -1),
      gp["W2"], gp["b2"].reshape(1, -1),
      gp["g"].reshape(1, -1), gp["beta"].reshape(1, -1))


def kernel(nodes, edges, globals_, edge_index, params):
    ei = edge_index.astype(jnp.int32)
    sidx = ei[:, 0].reshape(NW, NCH, CHUNK)
    didx = ei[:, 1].reshape(NW, NCH, CHUNK)
    src_feat, dst_feat = _gather_rows(nodes, sidx, didx)
    edges_new, esum = _edge_mlp(edges, src_feat, dst_feat, globals_,
                                params["edge"])
    zeros = jnp.zeros((N, DN), jnp.float32)
    agg2 = _scatter_add(edges_new, sidx, didx, zeros)
    nodes_new, globals_new = _node_mlp(nodes, agg2, globals_, esum,
                                       params["node"], params["global"])
    return nodes_new, edges_new, globals_new


# speedup vs baseline: 1.6342x; 1.0372x over previous
import functools

import jax
import jax.numpy as jnp
from jax import lax
from jax.experimental import pallas as pl
from jax.experimental.pallas import tpu as pltpu
from jax.experimental.pallas import tpu_sc as plsc

N = 10000
E = 160000
DN = 128
DG = 64

NC = 2
NS = 16
NW = NC * NS

CHUNK = 125
NSLAB = 4
SLAB_E = E // NSLAB
SLAB_PER_W = SLAB_E // NW
NCH_S = SLAB_PER_W // CHUNK

_gather_mesh = plsc.VectorSubcoreMesh(core_axis_name="c", subcore_axis_name="s")
_sc_params = pltpu.CompilerParams(use_tc_tiling_on_sc=False)


@functools.partial(
    pl.kernel,
    out_type=[jax.ShapeDtypeStruct((SLAB_E, DN), jnp.float32),
              jax.ShapeDtypeStruct((SLAB_E, DN), jnp.float32)],
    mesh=_gather_mesh,
    compiler_params=_sc_params,
    scratch_types=[
        pltpu.VMEM((NCH_S, CHUNK), jnp.int32),
        pltpu.VMEM((NCH_S, CHUNK), jnp.int32),
        pltpu.VMEM((CHUNK, DN), jnp.float32),
        pltpu.VMEM((CHUNK, DN), jnp.float32),
        pltpu.SemaphoreType.DMA,
        pltpu.SemaphoreType.DMA,
    ],
)
def _gather_rows(nodes_hbm, sidx_hbm, didx_hbm, sout_hbm, dout_hbm,
                 sidx_v, didx_v, srows_v, drows_v, sem1, sem2):
    wid = lax.axis_index("s") * NC + lax.axis_index("c")
    pltpu.sync_copy(sidx_hbm.at[wid], sidx_v)
    pltpu.sync_copy(didx_hbm.at[wid], didx_v)
    base = wid * SLAB_PER_W

    @pl.loop(0, NCH_S)
    def _chunk(c):
        cp1 = pltpu.async_copy(nodes_hbm.at[sidx_v.at[c]], srows_v, sem1)
        cp2 = pltpu.async_copy(nodes_hbm.at[didx_v.at[c]], drows_v, sem2)
        cp1.wait()
        pltpu.sync_copy(srows_v, sout_hbm.at[pl.ds(base + c * CHUNK, CHUNK)])
        cp2.wait()
        pltpu.sync_copy(drows_v, dout_hbm.at[pl.ds(base + c * CHUNK, CHUNK)])


S_ROWS_PER_TILE = N // NS


@functools.partial(
    pl.kernel,
    out_type=jax.ShapeDtypeStruct((NC, N, DN), jnp.float32),
    mesh=_gather_mesh,
    compiler_params=_sc_params,
    scratch_types=[
        pltpu.VMEM((NCH_S, CHUNK), jnp.int32),
        pltpu.VMEM((NCH_S, CHUNK), jnp.int32),
        pltpu.VMEM((CHUNK, DN), jnp.float32),
        pltpu.VMEM_SHARED((N, DN), jnp.float32),
    ],
)
def _scatter_add(feat_hbm, sidx_hbm, didx_hbm, zeros_hbm, out_hbm,
                 sidx_v, didx_v, feat_v, agg_sh):
    cid = lax.axis_index("c")
    sid = lax.axis_index("s")
    wid = sid * NC + cid
    rslice = pl.ds(sid * S_ROWS_PER_TILE, S_ROWS_PER_TILE)
    pltpu.sync_copy(zeros_hbm.at[rslice], agg_sh.at[rslice])
    plsc.subcore_barrier()

    for s in range(NSLAB):
        pltpu.sync_copy(sidx_hbm.at[s, wid], sidx_v)
        pltpu.sync_copy(didx_hbm.at[s, wid], didx_v)
        base = s * SLAB_E + wid * SLAB_PER_W

        @pl.loop(0, NCH_S)
        def _chunk(c):
            pltpu.sync_copy(feat_hbm.at[pl.ds(base + c * CHUNK, CHUNK)], feat_v)
            pltpu.sync_copy(feat_v, agg_sh.at[sidx_v.at[c]], add=True)
            pltpu.sync_copy(feat_v, agg_sh.at[didx_v.at[c]], add=True)

    plsc.subcore_barrier()
    pltpu.sync_copy(agg_sh.at[rslice], out_hbm.at[cid, rslice])


EB = 2000


def _edge_mlp_body(e_ref, s_ref, d_ref, gl_ref,
                   w0e, w0s, w0d, w0g, b0, w1, b1, w2, b2, gam, bet,
                   out_ref, sum_ref):
    i = pl.program_id(0)
    x = jnp.dot(e_ref[...].astype(jnp.bfloat16), w0e[...],
                preferred_element_type=jnp.float32)
    x += jnp.dot(s_ref[...].astype(jnp.bfloat16), w0s[...],
                 preferred_element_type=jnp.float32)
    x += jnp.dot(d_ref[...].astype(jnp.bfloat16), w0d[...],
                 preferred_element_type=jnp.float32)
    gbb = jnp.dot(gl_ref[...], w0g[...],
                  preferred_element_type=jnp.float32) + b0[...]
    h = jax.nn.relu(x + gbb)
    h = jax.nn.relu(jnp.dot(h.astype(jnp.bfloat16), w1[...],
                            preferred_element_type=jnp.float32) + b1[...])
    h = jax.nn.sigmoid(jnp.dot(h.astype(jnp.bfloat16), w2[...],
                               preferred_element_type=jnp.float32) + b2[...])
    mu = jnp.mean(h, axis=-1, keepdims=True)
    hc = h - mu
    var = jnp.mean(hc * hc, axis=-1, keepdims=True)
    y = hc * lax.rsqrt(var + 1e-3) * gam[...] + bet[...]
    out_ref[...] = y

    @pl.when(i == 0)
    def _init():
        sum_ref[...] = jnp.zeros_like(sum_ref)

    sum_ref[...] += jnp.sum(y, axis=0, keepdims=True)


def _edge_mlp_slab(slab, buf, edges, src_feat, dst_feat, globals_, p):
    w0 = p["W0"]

    def rep(w):
        return pl.BlockSpec(w.shape, lambda i: (0,) * w.ndim)

    steps = SLAB_E // EB
    off = slab * steps
    body = (_edge_mlp_body if buf is None
            else lambda buf_ref, *a: _edge_mlp_body(*a))
    return pl.pallas_call(
        body,
        grid=(steps,),
        in_specs=([pl.BlockSpec(memory_space=pl.ANY)] if buf is not None
                  else []) + [
            pl.BlockSpec((EB, 16), lambda i: (off + i, 0)),
            pl.BlockSpec((EB, 128), lambda i: (i, 0)),
            pl.BlockSpec((EB, 128), lambda i: (i, 0)),
            pl.BlockSpec((1, DG), lambda i: (0, 0)),
            rep(w0[:16]), rep(w0[16:144]), rep(w0[144:272]), rep(w0[272:336]),
            pl.BlockSpec((1, 256), lambda i: (0, 0)),
            pl.BlockSpec((256, 256), lambda i: (0, 0)),
            pl.BlockSpec((1, 256), lambda i: (0, 0)),
            pl.BlockSpec((256, 128), lambda i: (0, 0)),
            pl.BlockSpec((1, 128), lambda i: (0, 0)),
            pl.BlockSpec((1, 128), lambda i: (0, 0)),
            pl.BlockSpec((1, 128), lambda i: (0, 0)),
        ],
        out_specs=[
            pl.BlockSpec((EB, 128), lambda i: (off + i, 0)),
            pl.BlockSpec((1, 128), lambda i: (0, 0)),
        ],
        out_shape=[
            jax.ShapeDtypeStruct((E, 128), jnp.float32),
            jax.ShapeDtypeStruct((1, 128), jnp.float32),
        ],
        input_output_aliases={0: 0} if buf is not None else {},
    )(*(((buf,) if buf is not None else ()) + (edges, src_feat, dst_feat)),
      globals_.astype(jnp.bfloat16),
      w0[:16].astype(jnp.bfloat16), w0[16:144].astype(jnp.bfloat16),
      w0[144:272].astype(jnp.bfloat16), w0[272:336].astype(jnp.bfloat16),
      p["b0"].reshape(1, -1),
      p["W1"].astype(jnp.bfloat16), p["b1"].reshape(1, -1),
      p["W2"].astype(jnp.bfloat16), p["b2"].reshape(1, -1),
      p["g"].reshape(1, -1), p["beta"].reshape(1, -1))


NB = 2000


def _node_mlp_body(n_ref, a_ref, gl_ref, esum_ref,
                   w0n, w0g, w0a, b0, w1, b1, w2, b2, gam, bet,
                   gw0g, gw0n, gw0e, gb0, gw1, gb1, gw2, gb2, ggam, gbet,
                   out_ref, gout_ref, nsum):
    i = pl.program_id(0)
    agg = (a_ref[0] + a_ref[1]).astype(jnp.bfloat16)
    x = jnp.dot(n_ref[...], w0n[...], preferred_element_type=jnp.float32)
    x += jnp.dot(agg, w0a[...], preferred_element_type=jnp.float32)
    gbb = jnp.dot(gl_ref[...], w0g[...],
                  preferred_element_type=jnp.float32) + b0[...]
    h = jax.nn.relu(x + gbb)
    h = jax.nn.relu(jnp.dot(h.astype(jnp.bfloat16), w1[...],
                            preferred_element_type=jnp.float32) + b1[...])
    h = jax.nn.sigmoid(jnp.dot(h.astype(jnp.bfloat16), w2[...],
                               preferred_element_type=jnp.float32) + b2[...])
    mu = jnp.mean(h, axis=-1, keepdims=True)
    hc = h - mu
    var = jnp.mean(hc * hc, axis=-1, keepdims=True)
    y = hc * lax.rsqrt(var + 1e-3) * gam[...] + bet[...]
    out_ref[...] = y

    @pl.when(i == 0)
    def _init():
        nsum[...] = jnp.zeros_like(nsum)

    nsum[...] += jnp.sum(y, axis=0, keepdims=True)

    @pl.when(i == pl.num_programs(0) - 1)
    def _globals():
        nm = nsum[...] / N
        em = esum_ref[...] / E
        gx = jnp.dot(gl_ref[...], gw0g[...], preferred_element_type=jnp.float32)
        gx += jnp.dot(nm, gw0n[...], preferred_element_type=jnp.float32)
        gx += jnp.dot(em, gw0e[...], preferred_element_type=jnp.float32)
        gh = jax.nn.relu(gx + gb0[...])
        gh = jax.nn.relu(jnp.dot(gh, gw1[...], preferred_element_type=jnp.float32) + gb1[...])
        gh = jax.nn.sigmoid(jnp.dot(gh, gw2[...], preferred_element_type=jnp.float32) + gb2[...])
        gmu = jnp.mean(gh, axis=-1, keepdims=True)
        ghc = gh - gmu
        gvar = jnp.mean(ghc * ghc, axis=-1, keepdims=True)
        gout_ref[...] = ghc * lax.rsqrt(gvar + 1e-3) * ggam[...] + gbet[...]


def _node_mlp(nodes, agg2, globals_, esum, p, gp):
    w0 = p["W0"]
    gw0 = gp["W0"]

    def rep(w):
        return pl.BlockSpec(w.shape, lambda i: (0,) * w.ndim)

    grid = N // NB
    return pl.pallas_call(
        _node_mlp_body,
        grid=(grid,),
        in_specs=[
            pl.BlockSpec((NB, DN), lambda i: (i, 0)),
            pl.BlockSpec((NC, NB, DN), lambda i: (0, i, 0)),
            pl.BlockSpec((1, DG), lambda i: (0, 0)),
            pl.BlockSpec((1, 128), lambda i: (0, 0)),
            rep(w0[:128]), rep(w0[128:192]), rep(w0[192:320]),
            rep(p["b0"].reshape(1, -1)), rep(p["W1"]), rep(p["b1"].reshape(1, -1)),
            rep(p["W2"]), rep(p["b2"].reshape(1, -1)),
            rep(p["g"].reshape(1, -1)), rep(p["beta"].reshape(1, -1)),
            rep(gw0[:64]), rep(gw0[64:192]), rep(gw0[192:320]),
            rep(gp["b0"].reshape(1, -1)), rep(gp["W1"]), rep(gp["b1"].reshape(1, -1)),
            rep(gp["W2"]), rep(gp["b2"].reshape(1, -1)),
            rep(gp["g"].reshape(1, -1)), rep(gp["beta"].reshape(1, -1)),
        ],
        out_specs=[
            pl.BlockSpec((NB, 128), lambda i: (i, 0)),
            pl.BlockSpec((1, DG), lambda i: (0, 0)),
        ],
        out_shape=[
            jax.ShapeDtypeStruct((N, 128), jnp.float32),
            jax.ShapeDtypeStruct((1, DG), jnp.float32),
        ],
        scratch_shapes=[pltpu.VMEM((1, 128), jnp.float32)],
    )(nodes.astype(jnp.bfloat16), agg2, globals_, esum,
      w0[:128].astype(jnp.bfloat16), w0[128:192], w0[192:320].astype(jnp.bfloat16),
      p["b0"].reshape(1, -1), p["W1"].astype(jnp.bfloat16), p["b1"].reshape(1, -1),
      p["W2"].astype(jnp.bfloat16), p["b2"].reshape(1, -1),
      p["g"].reshape(1, -1), p["beta"].reshape(1, -1),
      gw0[:64], gw0[64:192], gw0[192:320],
      gp["b0"].reshape(1, -1), gp["W1"], gp["b1"].reshape(1, -1),
      gp["W2"], gp["b2"].reshape(1, -1),
      gp["g"].reshape(1, -1), gp["beta"].reshape(1, -1))


def kernel(nodes, edges, globals_, edge_index, params):
    ei = edge_index.astype(jnp.int32)
    sidx4 = ei[:, 0].reshape(NSLAB, NW, NCH_S, CHUNK)
    didx4 = ei[:, 1].reshape(NSLAB, NW, NCH_S, CHUNK)
    edges_new = None
    esum = None
    for s in range(NSLAB):
        sf, df = _gather_rows(nodes, sidx4[s], didx4[s])
        edges_new, es = _edge_mlp_slab(s, edges_new, edges, sf, df,
                                       globals_, params["edge"])
        esum = es if esum is None else esum + es
    zeros = jnp.zeros((N, DN), jnp.float32)
    agg2 = _scatter_add(edges_new, sidx4, didx4, zeros)
    nodes_new, globals_new = _node_mlp(nodes, agg2, globals_, esum,
                                       params["node"], params["global"])
    return nodes_new, edges_new, globals_new


# speedup vs baseline: 1.9173x; 1.1733x over previous
import functools

import jax
import jax.numpy as jnp
from jax import lax
from jax.experimental import pallas as pl
from jax.experimental.pallas import tpu as pltpu
from jax.experimental.pallas import tpu_sc as plsc

N = 10000
E = 160000
DN = 128
DG = 64

NC = 2
NS = 16
NW = NC * NS

CHUNK = 125
NSLAB = 4
SLAB_E = E // NSLAB
SLAB_PER_W = SLAB_E // NW
NCH_S = SLAB_PER_W // CHUNK

_gather_mesh = plsc.VectorSubcoreMesh(core_axis_name="c", subcore_axis_name="s")
_sc_params = pltpu.CompilerParams(use_tc_tiling_on_sc=False)


@functools.partial(
    pl.kernel,
    out_type=[jax.ShapeDtypeStruct((SLAB_E, DN), jnp.float32),
              jax.ShapeDtypeStruct((SLAB_E, DN), jnp.float32)],
    mesh=_gather_mesh,
    compiler_params=_sc_params,
    scratch_types=[
        pltpu.VMEM((NCH_S, CHUNK), jnp.int32),
        pltpu.VMEM((NCH_S, CHUNK), jnp.int32),
        pltpu.VMEM((CHUNK, DN), jnp.float32),
        pltpu.VMEM((CHUNK, DN), jnp.float32),
        pltpu.SemaphoreType.DMA,
        pltpu.SemaphoreType.DMA,
    ],
)
def _gather_rows(nodes_hbm, sidx_hbm, didx_hbm, sout_hbm, dout_hbm,
                 sidx_v, didx_v, srows_v, drows_v, sem1, sem2):
    wid = lax.axis_index("s") * NC + lax.axis_index("c")
    pltpu.sync_copy(sidx_hbm.at[wid], sidx_v)
    pltpu.sync_copy(didx_hbm.at[wid], didx_v)
    base = wid * SLAB_PER_W

    @pl.loop(0, NCH_S)
    def _chunk(c):
        cp1 = pltpu.async_copy(nodes_hbm.at[sidx_v.at[c]], srows_v, sem1)
        cp2 = pltpu.async_copy(nodes_hbm.at[didx_v.at[c]], drows_v, sem2)
        cp1.wait()
        pltpu.sync_copy(srows_v, sout_hbm.at[pl.ds(base + c * CHUNK, CHUNK)])
        cp2.wait()
        pltpu.sync_copy(drows_v, dout_hbm.at[pl.ds(base + c * CHUNK, CHUNK)])


S_ROWS_PER_TILE = N // NS


@functools.partial(
    pl.kernel,
    out_type=jax.ShapeDtypeStruct((NC, N, DN), jnp.float32),
    mesh=_gather_mesh,
    compiler_params=_sc_params,
    scratch_types=[
        pltpu.VMEM((NCH_S, CHUNK), jnp.int32),
        pltpu.VMEM((NCH_S, CHUNK), jnp.int32),
        pltpu.VMEM((CHUNK, DN), jnp.float32),
        pltpu.VMEM_SHARED((N, DN), jnp.float32),
    ],
)
def _scatter_add(feat_hbm, sidx_hbm, didx_hbm, zeros_hbm, out_hbm,
                 sidx_v, didx_v, feat_v, agg_sh):
    cid = lax.axis_index("c")
    sid = lax.axis_index("s")
    wid = sid * NC + cid
    rslice = pl.ds(sid * S_ROWS_PER_TILE, S_ROWS_PER_TILE)
    pltpu.sync_copy(zeros_hbm.at[rslice], agg_sh.at[rslice])
    pltpu.sync_copy(sidx_hbm.at[wid], sidx_v)
    pltpu.sync_copy(didx_hbm.at[wid], didx_v)
    plsc.subcore_barrier()
    base = wid * SLAB_PER_W

    @pl.loop(0, NCH_S)
    def _chunk(c):
        pltpu.sync_copy(feat_hbm.at[pl.ds(base + c * CHUNK, CHUNK)], feat_v)
        pltpu.sync_copy(feat_v, agg_sh.at[sidx_v.at[c]], add=True)
        pltpu.sync_copy(feat_v, agg_sh.at[didx_v.at[c]], add=True)

    plsc.subcore_barrier()
    pltpu.sync_copy(agg_sh.at[rslice], out_hbm.at[cid, rslice])


EB = 2000


def _edge_mlp_body(e_ref, s_ref, d_ref, gl_ref,
                   w0e, w0s, w0d, w0g, b0, w1, b1, w2, b2, gam, bet,
                   out_ref, slab_ref, sum_ref):
    i = pl.program_id(0)
    x = jnp.dot(e_ref[...], w0e[...], preferred_element_type=jnp.float32)
    x += jnp.dot(s_ref[...].astype(jnp.bfloat16), w0s[...],
                 preferred_element_type=jnp.float32)
    x += jnp.dot(d_ref[...].astype(jnp.bfloat16), w0d[...],
                 preferred_element_type=jnp.float32)
    gbb = jnp.dot(gl_ref[...], w0g[...],
                  preferred_element_type=jnp.float32) + b0[...]
    h = jax.nn.relu(x + gbb)
    h = jax.nn.relu(jnp.dot(h.astype(jnp.bfloat16), w1[...],
                            preferred_element_type=jnp.float32) + b1[...])
    h = jax.nn.sigmoid(jnp.dot(h.astype(jnp.bfloat16), w2[...],
                               preferred_element_type=jnp.float32) + b2[...])
    mu = jnp.mean(h, axis=-1, keepdims=True)
    hc = h - mu
    var = jnp.mean(hc * hc, axis=-1, keepdims=True)
    y = hc * lax.rsqrt(var + 1e-3) * gam[...] + bet[...]
    out_ref[...] = y
    slab_ref[...] = y

    @pl.when(i == 0)
    def _init():
        sum_ref[...] = jnp.zeros_like(sum_ref)

    sum_ref[...] += jnp.sum(y, axis=0, keepdims=True)


def _edge_mlp_slab(slab, buf, edges128, src_feat, dst_feat, globals_, p):
    w0 = p["W0"]

    def rep(w):
        return pl.BlockSpec(w.shape, lambda i: (0,) * w.ndim)

    steps = SLAB_E // EB
    off = slab * steps
    body = (_edge_mlp_body if buf is None
            else lambda buf_ref, *a: _edge_mlp_body(*a))
    return pl.pallas_call(
        body,
        grid=(steps,),
        in_specs=([pl.BlockSpec(memory_space=pl.ANY)] if buf is not None
                  else []) + [
            pl.BlockSpec((EB, 16), lambda i: (off + i, 0)),
            pl.BlockSpec((EB, 128), lambda i: (i, 0)),
            pl.BlockSpec((EB, 128), lambda i: (i, 0)),
            pl.BlockSpec((1, DG), lambda i: (0, 0)),
            rep(w0[:16]), rep(w0[16:144]), rep(w0[144:272]), rep(w0[272:336]),
            pl.BlockSpec((1, 256), lambda i: (0, 0)),
            pl.BlockSpec((256, 256), lambda i: (0, 0)),
            pl.BlockSpec((1, 256), lambda i: (0, 0)),
            pl.BlockSpec((256, 128), lambda i: (0, 0)),
            pl.BlockSpec((1, 128), lambda i: (0, 0)),
            pl.BlockSpec((1, 128), lambda i: (0, 0)),
            pl.BlockSpec((1, 128), lambda i: (0, 0)),
        ],
        out_specs=[
            pl.BlockSpec((EB, 128), lambda i: (off + i, 0)),
            pl.BlockSpec((EB, 128), lambda i: (i, 0)),
            pl.BlockSpec((1, 128), lambda i: (0, 0)),
        ],
        out_shape=[
            jax.ShapeDtypeStruct((E, 128), jnp.float32),
            jax.ShapeDtypeStruct((SLAB_E, 128), jnp.float32),
            jax.ShapeDtypeStruct((1, 128), jnp.float32),
        ],
        input_output_aliases={0: 0} if buf is not None else {},
    )(*(((buf,) if buf is not None else ()) + (edges128, src_feat, dst_feat)),
      globals_.astype(jnp.bfloat16),
      w0[:16].astype(jnp.bfloat16), w0[16:144].astype(jnp.bfloat16),
      w0[144:272].astype(jnp.bfloat16), w0[272:336].astype(jnp.bfloat16),
      p["b0"].reshape(1, -1),
      p["W1"].astype(jnp.bfloat16), p["b1"].reshape(1, -1),
      p["W2"].astype(jnp.bfloat16), p["b2"].reshape(1, -1),
      p["g"].reshape(1, -1), p["beta"].reshape(1, -1))


NB = 2000


def _node_mlp_body(n_ref, a0_ref, a1_ref, a2_ref, a3_ref, gl_ref, esum_ref,
                   w0n, w0g, w0a, b0, w1, b1, w2, b2, gam, bet,
                   gw0g, gw0n, gw0e, gb0, gw1, gb1, gw2, gb2, ggam, gbet,
                   out_ref, gout_ref, nsum):
    i = pl.program_id(0)
    agg = ((a0_ref[0] + a0_ref[1]) + (a1_ref[0] + a1_ref[1])
           + (a2_ref[0] + a2_ref[1]) + (a3_ref[0] + a3_ref[1])
           ).astype(jnp.bfloat16)
    x = jnp.dot(n_ref[...], w0n[...], preferred_element_type=jnp.float32)
    x += jnp.dot(agg, w0a[...], preferred_element_type=jnp.float32)
    gbb = jnp.dot(gl_ref[...], w0g[...],
                  preferred_element_type=jnp.float32) + b0[...]
    h = jax.nn.relu(x + gbb)
    h = jax.nn.relu(jnp.dot(h.astype(jnp.bfloat16), w1[...],
                            preferred_element_type=jnp.float32) + b1[...])
    h = jax.nn.sigmoid(jnp.dot(h.astype(jnp.bfloat16), w2[...],
                               preferred_element_type=jnp.float32) + b2[...])
    mu = jnp.mean(h, axis=-1, keepdims=True)
    hc = h - mu
    var = jnp.mean(hc * hc, axis=-1, keepdims=True)
    y = hc * lax.rsqrt(var + 1e-3) * gam[...] + bet[...]
    out_ref[...] = y

    @pl.when(i == 0)
    def _init():
        nsum[...] = jnp.zeros_like(nsum)

    nsum[...] += jnp.sum(y, axis=0, keepdims=True)

    @pl.when(i == pl.num_programs(0) - 1)
    def _globals():
        nm = nsum[...] / N
        em = esum_ref[...] / E
        gx = jnp.dot(gl_ref[...], gw0g[...], preferred_element_type=jnp.float32)
        gx += jnp.dot(nm, gw0n[...], preferred_element_type=jnp.float32)
        gx += jnp.dot(em, gw0e[...], preferred_element_type=jnp.float32)
        gh = jax.nn.relu(gx + gb0[...])
        gh = jax.nn.relu(jnp.dot(gh, gw1[...], preferred_element_type=jnp.float32) + gb1[...])
        gh = jax.nn.sigmoid(jnp.dot(gh, gw2[...], preferred_element_type=jnp.float32) + gb2[...])
        gmu = jnp.mean(gh, axis=-1, keepdims=True)
        ghc = gh - gmu
        gvar = jnp.mean(ghc * ghc, axis=-1, keepdims=True)
        gout_ref[...] = ghc * lax.rsqrt(gvar + 1e-3) * ggam[...] + gbet[...]


def _node_mlp(nodes, aggs, globals_, esum, p, gp):
    w0 = p["W0"]
    gw0 = gp["W0"]

    def rep(w):
        return pl.BlockSpec(w.shape, lambda i: (0,) * w.ndim)

    grid = N // NB
    return pl.pallas_call(
        _node_mlp_body,
        grid=(grid,),
        in_specs=[
            pl.BlockSpec((NB, DN), lambda i: (i, 0)),
            pl.BlockSpec((NC, NB, DN), lambda i: (0, i, 0)),
            pl.BlockSpec((NC, NB, DN), lambda i: (0, i, 0)),
            pl.BlockSpec((NC, NB, DN), lambda i: (0, i, 0)),
            pl.BlockSpec((NC, NB, DN), lambda i: (0, i, 0)),
            pl.BlockSpec((1, DG), lambda i: (0, 0)),
            pl.BlockSpec((1, 128), lambda i: (0, 0)),
            rep(w0[:128]), rep(w0[128:192]), rep(w0[192:320]),
            rep(p["b0"].reshape(1, -1)), rep(p["W1"]), rep(p["b1"].reshape(1, -1)),
            rep(p["W2"]), rep(p["b2"].reshape(1, -1)),
            rep(p["g"].reshape(1, -1)), rep(p["beta"].reshape(1, -1)),
            rep(gw0[:64]), rep(gw0[64:192]), rep(gw0[192:320]),
            rep(gp["b0"].reshape(1, -1)), rep(gp["W1"]), rep(gp["b1"].reshape(1, -1)),
            rep(gp["W2"]), rep(gp["b2"].reshape(1, -1)),
            rep(gp["g"].reshape(1, -1)), rep(gp["beta"].reshape(1, -1)),
        ],
        out_specs=[
            pl.BlockSpec((NB, 128), lambda i: (i, 0)),
            pl.BlockSpec((1, DG), lambda i: (0, 0)),
        ],
        out_shape=[
            jax.ShapeDtypeStruct((N, 128), jnp.float32),
            jax.ShapeDtypeStruct((1, DG), jnp.float32),
        ],
        scratch_shapes=[pltpu.VMEM((1, 128), jnp.float32)],
    )(nodes.astype(jnp.bfloat16), aggs[0], aggs[1], aggs[2], aggs[3],
      globals_, esum,
      w0[:128].astype(jnp.bfloat16), w0[128:192], w0[192:320].astype(jnp.bfloat16),
      p["b0"].reshape(1, -1), p["W1"].astype(jnp.bfloat16), p["b1"].reshape(1, -1),
      p["W2"].astype(jnp.bfloat16), p["b2"].reshape(1, -1),
      p["g"].reshape(1, -1), p["beta"].reshape(1, -1),
      gw0[:64], gw0[64:192], gw0[192:320],
      gp["b0"].reshape(1, -1), gp["W1"], gp["b1"].reshape(1, -1),
      gp["W2"], gp["b2"].reshape(1, -1),
      gp["g"].reshape(1, -1), gp["beta"].reshape(1, -1))


def kernel(nodes, edges, globals_, edge_index, params):
    ei = edge_index.astype(jnp.int32)
    sidx4 = ei[:, 0].reshape(NSLAB, NW, NCH_S, CHUNK)
    didx4 = ei[:, 1].reshape(NSLAB, NW, NCH_S, CHUNK)
    edges128 = edges.astype(jnp.bfloat16)
    zeros = jnp.zeros((N, DN), jnp.float32)
    edges_new = None
    esum = None
    aggs = []
    for s in range(NSLAB):
        sf, df = _gather_rows(nodes, sidx4[s], didx4[s])
        edges_new, en_s, es = _edge_mlp_slab(s, edges_new, edges128, sf, df,
                                             globals_, params["edge"])
        esum = es if esum is None else esum + es
        aggs.append(_scatter_add(en_s, sidx4[s], didx4[s], zeros))
    nodes_new, globals_new = _node_mlp(nodes, aggs, globals_, esum,
                                       params["node"], params["global"])
    return nodes_new, edges_new, globals_new
